# baseline (speedup 1.0000x reference)
import math

import jax
import jax.numpy as jnp
from jax import lax
from jax.experimental import pallas as pl
from jax.experimental.pallas import tpu as pltpu
from jax.experimental.pallas import tpu_sc as plsc

N = 10000
NP = 10240
E = 320000
H = 128
HEADS = 8
DKH = 16
MAXD = 240
DP = 256
G = 512
GP = 640
NF = 9
VOCAB = 128

NC = 2
NS = 16
NW = NC * NS

C = 16
CPT = 640
EP = NW * CPT * C
IDXG = 16
GROUPS = CPT // IDXG

NODES_PT = NP // NW
AC = 80
ACH = NODES_PT // AC

BR = 1280

_f32 = jnp.float32
_i32 = jnp.int32


def _mesh():
    return plsc.VectorSubcoreMesh(
        core_axis_name="c", subcore_axis_name="s", num_cores=NC, num_subcores=NS)


_GDN = lax.GatherDimensionNumbers(
    offset_dims=(), collapsed_slice_dims=(0,), start_index_map=(0,))


def _vgather(x, idx):
    return lax.gather(x, idx[:, None], _GDN, (1,),
                      mode=lax.GatherScatterMode.PROMISE_IN_BOUNDS)


def _hsum16(x):
    lane = lax.iota(_i32, 16)
    for sh in (8, 4, 2, 1):
        x = x + _vgather(x, lane ^ sh)
    return x


def _atom_body(emb_hbm, aidx_hbm, out_hbm, idxv, bufs, sem):
    c = lax.axis_index("c")
    s = lax.axis_index("s")
    wid = s * NC + c

    @pl.loop(0, ACH)
    def _chunk(ci):
        chunk = wid * ACH + ci
        base = wid * NODES_PT + ci * AC
        pltpu.sync_copy(aidx_hbm.at[chunk], idxv)
        cps = [pltpu.async_copy(emb_hbm.at[idxv.at[f]], bufs.at[f], sem)
               for f in range(NF)]
        for cp in cps:
            cp.wait()

        @pl.loop(0, AC)
        def _row(r):
            for g in range(H // 16):
                sl = pl.ds(g * 16, 16)
                acc = bufs[0, r, sl]
                for f in range(1, NF):
                    acc = acc + bufs[f, r, sl]
                bufs[0, r, sl] = acc

        pltpu.sync_copy(bufs.at[0], out_hbm.at[pl.ds(base, AC)])


_atom_call = pl.kernel(
    _atom_body,
    out_type=jax.ShapeDtypeStruct((NP, H), _f32),
    mesh=_mesh(),
    scratch_types=[
        pltpu.VMEM((NF, AC), _i32),
        pltpu.VMEM((NF, AC, H), _f32),
        pltpu.SemaphoreType.DMA,
    ],
)


def _edge_body(nq_hbm, nkv_hbm, dkv_hbm, src_hbm, dst_hbm, str_hbm, zeros_hbm,
               num_hbm, den_hbm, accn, accd, idxs, idxd, idxt,
               qb0, kvb0, dkb0, ob0, db0, dib0,
               qb1, kvb1, dkb1, ob1, db1, dib1,
               sq0, sk0, sd0, sn0, sm0, sq1, sk1, sd1, sn1, sm1):
    c = lax.axis_index("c")
    s = lax.axis_index("s")
    wid = s * NC + c
    rpn = NP // NS
    rpd = (NP // 8) // NS

    pltpu.sync_copy(zeros_hbm.at[pl.ds(s * rpn, rpn)],
                    accn.at[pl.ds(s * rpn, rpn)])
    pltpu.sync_copy(zeros_hbm.at[pl.ds(s * rpd, rpd)],
                    accd.at[pl.ds(s * rpd, rpd)])
    plsc.subcore_barrier()

    lane = lax.iota(_i32, 16)
    headmask = jnp.where(lane < 8, 1.0, 0.0).astype(_f32)

    def _issue(cr, qb, kvb, dkb, sq, sk, sd):
        pltpu.async_copy(nq_hbm.at[idxd.at[cr]], qb, sq)
        pltpu.async_copy(nkv_hbm.at[idxs.at[cr]], kvb, sk)
        pltpu.async_copy(dkv_hbm.at[idxt.at[cr]], dkb, sd)

    def _wait_gather(qb, kvb, dkb, sq, sk, sd):
        pltpu.make_async_copy(nq_hbm.at[idxd.at[0]], qb, sq).wait()
        pltpu.make_async_copy(nkv_hbm.at[idxs.at[0]], kvb, sk).wait()
        pltpu.make_async_copy(dkv_hbm.at[idxt.at[0]], dkb, sd).wait()

    def _drain_scatter(ob, db, dib, sn, sm):
        pltpu.make_async_copy(ob, accn.at[idxd.at[0]], sn).wait()
        pltpu.make_async_copy(db, accd.at[dib], sm).wait()

    def _compute(cr, qb, kvb, dkb, ob, db, dib):
        dvec = idxd[cr, pl.ds(0, 16)]
        d7f = (dvec & 7).astype(_f32)
        dib[pl.ds(0, 16)] = lax.shift_right_logical(dvec, 3)

        @pl.loop(0, C, unroll=8)
        def _edge(e):
            sel = []
            for h in range(HEADS):
                sl = pl.ds(h * 16, 16)
                p = qb[e, sl] * (kvb[e, sl] + dkb[e, sl])
                sel.append(jnp.where(lane == h, _hsum16(p), 0.0))
            sp = ((sel[0] + sel[1]) + (sel[2] + sel[3])) + (
                (sel[4] + sel[5]) + (sel[6] + sel[7]))
            ev = jnp.exp(sp * 0.25) * headmask
            for h in range(HEADS):
                sl = pl.ds(h * 16, 16)
                sv = pl.ds(H + h * 16, 16)
                evh = _vgather(ev, jnp.full((16,), h, _i32))
                ob[e, sl] = (kvb[e, sv] + dkb[e, sv]) * evh
            dsp = _vgather(d7f, jnp.broadcast_to(e, (16,)))
            for g2 in range(8):
                m = jnp.maximum(0.0, 1.0 - jnp.abs(dsp - float(g2)))
                db[e, pl.ds(g2 * 16, 16)] = ev * m

    def _scatter(cr, ob, db, dib, sn, sm):
        pltpu.async_copy(ob, accn.at[idxd.at[cr]], sn, add=True)
        pltpu.async_copy(db, accd.at[dib], sm, add=True)

    @pl.loop(0, GROUPS)
    def _group(g):
        @pl.when(g > 0)
        def _gd():
            _drain_scatter(ob0, db0, dib0, sn0, sm0)
            _drain_scatter(ob1, db1, dib1, sn1, sm1)
        row0 = wid * CPT + g * IDXG
        pltpu.sync_copy(src_hbm.at[pl.ds(row0, IDXG)], idxs)
        pltpu.sync_copy(dst_hbm.at[pl.ds(row0, IDXG)], idxd)
        pltpu.sync_copy(str_hbm.at[pl.ds(row0, IDXG)], idxt)
        _issue(0, qb0, kvb0, dkb0, sq0, sk0, sd0)

        @pl.loop(0, IDXG // 2)
        def _pair(jp):
            a = 2 * jp
            _issue(a + 1, qb1, kvb1, dkb1, sq1, sk1, sd1)
            _wait_gather(qb0, kvb0, dkb0, sq0, sk0, sd0)

            @pl.when(jp > 0)
            def _d0():
                _drain_scatter(ob0, db0, dib0, sn0, sm0)
            _compute(a, qb0, kvb0, dkb0, ob0, db0, dib0)
            _scatter(a, ob0, db0, dib0, sn0, sm0)

            @pl.when(jp < IDXG // 2 - 1)
            def _pf():
                _issue(a + 2, qb0, kvb0, dkb0, sq0, sk0, sd0)
            _wait_gather(qb1, kvb1, dkb1, sq1, sk1, sd1)

            @pl.when(jp > 0)
            def _d1():
                _drain_scatter(ob1, db1, dib1, sn1, sm1)
            _compute(a + 1, qb1, kvb1, dkb1, ob1, db1, dib1)
            _scatter(a + 1, ob1, db1, dib1, sn1, sm1)

    _drain_scatter(ob0, db0, dib0, sn0, sm0)
    _drain_scatter(ob1, db1, dib1, sn1, sm1)
    plsc.subcore_barrier()
    pltpu.sync_copy(accn.at[pl.ds(s * rpn, rpn)],
                    num_hbm.at[c, pl.ds(s * rpn, rpn)])
    pltpu.sync_copy(accd.at[pl.ds(s * rpd, rpd)],
                    den_hbm.at[c, pl.ds(s * rpd, rpd)])


_edge_call = pl.kernel(
    _edge_body,
    out_type=(
        jax.ShapeDtypeStruct((NC, NP, H), _f32),
        jax.ShapeDtypeStruct((NC, NP // 8, H), _f32),
    ),
    mesh=_mesh(),
    scratch_types=[
        pltpu.VMEM_SHARED((NP, H), _f32),
        pltpu.VMEM_SHARED((NP // 8, H), _f32),
        pltpu.VMEM((IDXG, C), _i32),
        pltpu.VMEM((IDXG, C), _i32),
        pltpu.VMEM((IDXG, C), _i32),
        pltpu.VMEM((C, H), _f32),
        pltpu.VMEM((C, 2 * H), _f32),
        pltpu.VMEM((C, 2 * H), _f32),
        pltpu.VMEM((C, H), _f32),
        pltpu.VMEM((C, H), _f32),
        pltpu.VMEM((C,), _i32),
        pltpu.VMEM((C, H), _f32),
        pltpu.VMEM((C, 2 * H), _f32),
        pltpu.VMEM((C, 2 * H), _f32),
        pltpu.VMEM((C, H), _f32),
        pltpu.VMEM((C, H), _f32),
        pltpu.VMEM((C,), _i32),
        pltpu.SemaphoreType.DMA,
        pltpu.SemaphoreType.DMA,
        pltpu.SemaphoreType.DMA,
        pltpu.SemaphoreType.DMA,
        pltpu.SemaphoreType.DMA,
        pltpu.SemaphoreType.DMA,
        pltpu.SemaphoreType.DMA,
        pltpu.SemaphoreType.DMA,
        pltpu.SemaphoreType.DMA,
        pltpu.SemaphoreType.DMA,
    ],
)


def _pool_body(nr_hbm, bidx_hbm, zeros_hbm, sum_hbm, cnt_hbm,
               accs, accc, idxb, dib, nbuf, db):
    c = lax.axis_index("c")
    s = lax.axis_index("s")
    wid = s * NC + c
    rps = GP // NS
    rpc = 8

    pltpu.sync_copy(zeros_hbm.at[pl.ds(s * rps, rps)],
                    accs.at[pl.ds(s * rps, rps)])

    @pl.when(s < (GP // 8) // rpc)
    def _zc():
        pltpu.sync_copy(zeros_hbm.at[pl.ds(s * rpc, rpc)],
                        accc.at[pl.ds(s * rpc, rpc)])
    plsc.subcore_barrier()

    lane = lax.iota(_i32, 16)
    marker = jnp.where(lane == 0, 1.0, 0.0).astype(_f32)
    zero16 = jnp.zeros((16,), _f32)

    @pl.loop(0, ACH)
    def _chunk(ci):
        chunk = wid * ACH + ci
        base = wid * NODES_PT + ci * AC
        pltpu.sync_copy(bidx_hbm.at[pl.ds(chunk, 1)], idxb)
        pltpu.sync_copy(nr_hbm.at[pl.ds(base, AC)], nbuf)

        @pl.loop(0, AC)
        def _row(r):
            bvec = idxb[0, pl.ds((r >> 4) << 4, 16)]
            b7f = (bvec & 7).astype(_f32)
            bsp = _vgather(b7f, jnp.broadcast_to(r & 15, (16,)))
            for g2 in range(8):
                m = jnp.maximum(0.0, 1.0 - jnp.abs(bsp - float(g2)))
                db[r, pl.ds(g2 * 16, 16)] = marker * m

        for t in range(AC // 16):
            sl = pl.ds(t * 16, 16)
            dib[sl] = lax.shift_right_logical(idxb[0, sl], 3)

        pltpu.sync_copy(nbuf, accs.at[idxb.at[0]], add=True)
        pltpu.sync_copy(db, accc.at[dib], add=True)

    plsc.subcore_barrier()
    pltpu.sync_copy(accs.at[pl.ds(s * rps, rps)],
                    sum_hbm.at[c, pl.ds(s * rps, rps)])

    @pl.when(s < (GP // 8) // rpc)
    def _dc():
        pltpu.sync_copy(accc.at[pl.ds(s * rpc, rpc)],
                        cnt_hbm.at[c, pl.ds(s * rpc, rpc)])


_pool_call = pl.kernel(
    _pool_body,
    out_type=(
        jax.ShapeDtypeStruct((NC, GP, H), _f32),
        jax.ShapeDtypeStruct((NC, GP // 8, H), _f32),
    ),
    mesh=_mesh(),
    scratch_types=[
        pltpu.VMEM_SHARED((GP, H), _f32),
        pltpu.VMEM_SHARED((GP // 8, H), _f32),
        pltpu.VMEM((1, AC), _i32),
        pltpu.VMEM((AC,), _i32),
        pltpu.VMEM((AC, H), _f32),
        pltpu.VMEM((AC, H), _f32),
    ],
)


def _dot(a, b):
    return lax.dot_general(a, b, (((1,), (0,)), ((), ())),
                           precision=lax.Precision.HIGHEST,
                           preferred_element_type=_f32)


def _gelu(x):
    return 0.5 * x * (1.0 + lax.erf(x * (1.0 / math.sqrt(2.0))))


def _ln(x, g, b):
    mu = jnp.mean(x, axis=-1, keepdims=True)
    xc = x - mu
    var = jnp.mean(xc * xc, axis=-1, keepdims=True)
    return xc * lax.rsqrt(var + 1e-5) * g + b


def _qkv_body(x_ref, w_ref, bq_ref, nq_ref, nkv_ref):
    y = _dot(x_ref[...], w_ref[...])
    nq_ref[...] = y[:, :H] + bq_ref[...]
    nkv_ref[...] = y[:, H:]


def _qkv_call(x, w3, bq):
    return pl.pallas_call(
        _qkv_body,
        grid=(NP // BR,),
        in_specs=[
            pl.BlockSpec((BR, H), lambda i: (i, 0)),
            pl.BlockSpec((H, 3 * H), lambda i: (0, 0)),
            pl.BlockSpec((1, H), lambda i: (0, 0)),
        ],
        out_specs=[
            pl.BlockSpec((BR, H), lambda i: (i, 0)),
            pl.BlockSpec((BR, 2 * H), lambda i: (i, 0)),
        ],
        out_shape=[
            jax.ShapeDtypeStruct((NP, H), _f32),
            jax.ShapeDtypeStruct((NP, 2 * H), _f32),
        ],
    )(x, w3, bq)


def _dist_body(d_ref, wk_ref, wv_ref, bk_ref, bv_ref, out_ref):
    d = d_ref[...]
    dk = _dot(d, wk_ref[...]) + bk_ref[...]
    dv = _dot(d, wv_ref[...]) + bv_ref[...]
    out_ref[...] = jnp.concatenate([dk, dv], axis=1)


def _dist_call(demb, wk, wv, bk, bv):
    return pl.pallas_call(
        _dist_body,
        out_shape=jax.ShapeDtypeStruct((DP, 2 * H), _f32),
    )(demb, wk, wv, bk, bv)


def _upd_body(num_ref, den_ref, nr_ref, wa_ref, ba_ref, g1_ref, b1_ref,
              wm_ref, bm_ref, wo_ref, bo_ref, g2_ref, b2_ref, out_ref):
    num = num_ref[0] + num_ref[1]
    den8 = den_ref[0][:, :8] + den_ref[1][:, :8]
    r8 = lax.broadcasted_iota(_i32, (8, H), 0)
    c128 = lax.broadcasted_iota(_i32, (8, H), 1) // DKH
    sel = (r8 == c128).astype(_f32)
    den = _dot(den8, sel)
    aggr = num / (den + 1e-16)
    a = _dot(_gelu(aggr), wa_ref[...]) + ba_ref[...] + nr_ref[...]
    trans = _ln(a, g1_ref[...], b1_ref[...])
    mid = _gelu(_dot(trans, wm_ref[...]) + bm_ref[...])
    nr2 = _ln(_dot(mid, wo_ref[...]) + bo_ref[...] + trans,
              g2_ref[...], b2_ref[...])
    rows = pl.program_id(0) * BR + lax.broadcasted_iota(_i32, (BR, H), 0)
    out_ref[...] = jnp.where(rows < N, nr2, 0.0)


def _upd_call(num, den, nr, wa, ba, g1, b1, wm, bm, wo, bo, g2, b2):
    vec = lambda i: (0, 0)
    return pl.pallas_call(
        _upd_body,
        grid=(NP // BR,),
        in_specs=[
            pl.BlockSpec((2, BR, H), lambda i: (0, i, 0)),
            pl.BlockSpec((2, BR, 16), lambda i: (0, i, 0)),
            pl.BlockSpec((BR, H), lambda i: (i, 0)),
            pl.BlockSpec((H, H), vec),
            pl.BlockSpec((1, H), vec),
            pl.BlockSpec((1, H), vec),
            pl.BlockSpec((1, H), vec),
            pl.BlockSpec((H, 2 * H), vec),
            pl.BlockSpec((1, 2 * H), vec),
            pl.BlockSpec((2 * H, H), vec),
            pl.BlockSpec((1, H), vec),
            pl.BlockSpec((1, H), vec),
            pl.BlockSpec((1, H), vec),
        ],
        out_specs=pl.BlockSpec((BR, H), lambda i: (i, 0)),
        out_shape=jax.ShapeDtypeStruct((NP, H), _f32),
    )(num, den, nr, wa, ba, g1, b1, wm, bm, wo, bo, g2, b2)


def _final_body(ps_ref, pc_ref, wout_ref, bout_ref, out_ref):
    sums = ps_ref[0][:G, :] + ps_ref[1][:G, :]
    cnt = pc_ref[0][:G, :1] + pc_ref[1][:G, :1]
    mean = sums / jnp.maximum(cnt, 1.0)
    out_ref[...] = _dot(mean, wout_ref[...]) + bout_ref[...]


def _final_call(psum, pcnt, wout, bout):
    return pl.pallas_call(
        _final_body,
        out_shape=jax.ShapeDtypeStruct((G, H), _f32),
    )(psum, pcnt, wout, bout)


def kernel(node_attr, batch_idx, edge_index, strats, params):
    p = params
    attr = node_attr.astype(_i32)
    attr_p = jnp.pad(attr, ((0, NP - N), (0, 0)))
    aidx = (attr_p + (jnp.arange(NF, dtype=_i32) * VOCAB)[None, :]).T
    aidx = aidx.reshape(NF, NW * ACH, AC).transpose(1, 0, 2)
    emb2d = p['atom_emb'].reshape(NF * VOCAB, H)

    node_rep = _atom_call(emb2d, aidx)

    src = edge_index[0].astype(_i32)
    dst = edge_index[1].astype(_i32)
    st = strats.astype(_i32)
    padn = EP - E
    srcp = jnp.concatenate([src, jnp.zeros((padn,), _i32)]).reshape(EP // C, C)
    dstp = jnp.concatenate([dst, jnp.full((padn,), NP - 1, _i32)]).reshape(EP // C, C)
    stp = jnp.concatenate([st, jnp.zeros((padn,), _i32)]).reshape(EP // C, C)
    zeros = jnp.zeros((NP, H), _f32)

    for l in range(2):
        w3 = jnp.concatenate([p['Wq'][l], p['Wk'][l], p['Wv'][l]], axis=1)
        nq, nkv = _qkv_call(node_rep, w3, p['bq'][l].reshape(1, H))
        demb = jnp.pad(p['dist_emb'][l], ((0, DP - MAXD), (0, 0)))
        dkv = _dist_call(demb, p['Wk'][l], p['Wv'][l],
                         p['bk'][l].reshape(1, H), p['bv'][l].reshape(1, H))
        num, den = _edge_call(nq, nkv, dkv, srcp, dstp, stp, zeros)
        den = den.reshape(NC, NP, 16)
        node_rep = _upd_call(
            num, den, node_rep,
            p['Wa'][l], p['ba'][l].reshape(1, H),
            p['ln1_g'][l].reshape(1, H), p['ln1_b'][l].reshape(1, H),
            p['Wmid'][l], p['bmid'][l].reshape(1, 2 * H),
            p['Wo2'][l], p['bo2'][l].reshape(1, H),
            p['ln2_g'][l].reshape(1, H), p['ln2_b'][l].reshape(1, H))

    bidx = jnp.concatenate([batch_idx.astype(_i32), jnp.full((NP - N,), G, _i32)])
    bidx = bidx.reshape(NW * ACH, AC)
    psum, pcnt = _pool_call(node_rep, bidx, zeros)
    pcnt = pcnt.reshape(NC, GP, 16)
    return _final_call(psum, pcnt, p['Wout'], p['bout'].reshape(1, H))


# speedup vs baseline: 1.2075x; 1.2075x over previous
import math

import jax
import jax.numpy as jnp
from jax import lax
from jax.experimental import pallas as pl
from jax.experimental.pallas import tpu as pltpu
from jax.experimental.pallas import tpu_sc as plsc

N = 10000
NP = 10240
E = 320000
H = 128
HEADS = 8
DKH = 16
MAXD = 240
DP = 256
G = 512
GP = 640
NF = 9
VOCAB = 128

NC = 2
NS = 16
NW = NC * NS

C = 16
CPT = 640
EP = NW * CPT * C
IDXG = 16
GROUPS = CPT // IDXG

NODES_PT = NP // NW
AC = 80
ACH = NODES_PT // AC

BR = 1280

_f32 = jnp.float32
_i32 = jnp.int32


def _mesh():
    return plsc.VectorSubcoreMesh(
        core_axis_name="c", subcore_axis_name="s", num_cores=NC, num_subcores=NS)


_GDN = lax.GatherDimensionNumbers(
    offset_dims=(), collapsed_slice_dims=(0,), start_index_map=(0,))


def _vgather(x, idx):
    return lax.gather(x, idx[:, None], _GDN, (1,),
                      mode=lax.GatherScatterMode.PROMISE_IN_BOUNDS)


def _hsum16(x):
    lane = lax.iota(_i32, 16)
    for sh in (8, 4, 2, 1):
        x = x + _vgather(x, lane ^ sh)
    return x


def _atom_body(emb_hbm, aidx_hbm, out_hbm, idxv, bufs, sem):
    c = lax.axis_index("c")
    s = lax.axis_index("s")
    wid = s * NC + c

    @pl.loop(0, ACH)
    def _chunk(ci):
        chunk = wid * ACH + ci
        base = wid * NODES_PT + ci * AC
        pltpu.sync_copy(aidx_hbm.at[chunk], idxv)
        cps = [pltpu.async_copy(emb_hbm.at[idxv.at[f]], bufs.at[f], sem)
               for f in range(NF)]
        for cp in cps:
            cp.wait()

        @pl.loop(0, AC)
        def _row(r):
            for g in range(H // 16):
                sl = pl.ds(g * 16, 16)
                acc = bufs[0, r, sl]
                for f in range(1, NF):
                    acc = acc + bufs[f, r, sl]
                bufs[0, r, sl] = acc

        pltpu.sync_copy(bufs.at[0], out_hbm.at[pl.ds(base, AC)])


_atom_call = pl.kernel(
    _atom_body,
    out_type=jax.ShapeDtypeStruct((NP, H), _f32),
    mesh=_mesh(),
    scratch_types=[
        pltpu.VMEM((NF, AC), _i32),
        pltpu.VMEM((NF, AC, H), _f32),
        pltpu.SemaphoreType.DMA,
    ],
)


def _edge_body(nq_hbm, nkv_hbm, dkv_hbm, src_hbm, dst_hbm, str_hbm, zeros_hbm,
               num_hbm, den_hbm, accn, accd, idxs, idxd, idxt,
               qb0, kvb0, dkb0, ob0, db0, dib0,
               qb1, kvb1, dkb1, ob1, db1, dib1,
               sq0, sk0, sd0, sn0, sm0, sq1, sk1, sd1, sn1, sm1):
    c = lax.axis_index("c")
    s = lax.axis_index("s")
    wid = s * NC + c
    rpn = NP // NS
    rpd = (NP // 8) // NS

    pltpu.sync_copy(zeros_hbm.at[pl.ds(s * rpn, rpn)],
                    accn.at[pl.ds(s * rpn, rpn)])
    pltpu.sync_copy(zeros_hbm.at[pl.ds(s * rpd, rpd)],
                    accd.at[pl.ds(s * rpd, rpd)])
    plsc.subcore_barrier()

    lane = lax.iota(_i32, 16)
    headmask = jnp.where(lane < 8, 1.0, 0.0).astype(_f32)

    def _issue(cr, qb, kvb, dkb, sq, sk, sd):
        pltpu.async_copy(nq_hbm.at[idxd.at[cr]], qb, sq)
        pltpu.async_copy(nkv_hbm.at[idxs.at[cr]], kvb, sk)
        pltpu.async_copy(dkv_hbm.at[idxt.at[cr]], dkb, sd)

    def _wait_gather(qb, kvb, dkb, sq, sk, sd):
        pltpu.make_async_copy(nq_hbm.at[idxd.at[0]], qb, sq).wait()
        pltpu.make_async_copy(nkv_hbm.at[idxs.at[0]], kvb, sk).wait()
        pltpu.make_async_copy(dkv_hbm.at[idxt.at[0]], dkb, sd).wait()

    def _drain_scatter(ob, db, dib, sn, sm):
        pltpu.make_async_copy(ob, accn.at[idxd.at[0]], sn).wait()
        pltpu.make_async_copy(db, accd.at[dib], sm).wait()

    def _compute(cr, qb, kvb, dkb, ob, db, dib):
        dvec = idxd[cr, pl.ds(0, 16)]
        d7f = (dvec & 7).astype(_f32)
        dib[pl.ds(0, 16)] = lax.shift_right_logical(dvec, 3)

        @pl.loop(0, C, unroll=2)
        def _edge(e):
            sel = []
            for h in range(HEADS):
                sl = pl.ds(h * 16, 16)
                p = qb[e, sl] * (kvb[e, sl] + dkb[e, sl])
                sel.append(jnp.where(lane == h, _hsum16(p), 0.0))
            sp = ((sel[0] + sel[1]) + (sel[2] + sel[3])) + (
                (sel[4] + sel[5]) + (sel[6] + sel[7]))
            ev = jnp.exp(sp * 0.25) * headmask
            for h in range(HEADS):
                sl = pl.ds(h * 16, 16)
                sv = pl.ds(H + h * 16, 16)
                evh = _vgather(ev, jnp.full((16,), h, _i32))
                ob[e, sl] = (kvb[e, sv] + dkb[e, sv]) * evh
            dsp = _vgather(d7f, jnp.broadcast_to(e, (16,)))
            for g2 in range(8):
                m = jnp.maximum(0.0, 1.0 - jnp.abs(dsp - float(g2)))
                db[e, pl.ds(g2 * 16, 16)] = ev * m

    def _scatter(cr, ob, db, dib, sn, sm):
        pltpu.async_copy(ob, accn.at[idxd.at[cr]], sn, add=True)
        pltpu.async_copy(db, accd.at[dib], sm, add=True)

    @pl.loop(0, GROUPS)
    def _group(g):
        @pl.when(g > 0)
        def _gd():
            _drain_scatter(ob0, db0, dib0, sn0, sm0)
            _drain_scatter(ob1, db1, dib1, sn1, sm1)
        row0 = wid * CPT + g * IDXG
        pltpu.sync_copy(src_hbm.at[pl.ds(row0, IDXG)], idxs)
        pltpu.sync_copy(dst_hbm.at[pl.ds(row0, IDXG)], idxd)
        pltpu.sync_copy(str_hbm.at[pl.ds(row0, IDXG)], idxt)
        _issue(0, qb0, kvb0, dkb0, sq0, sk0, sd0)

        @pl.loop(0, IDXG // 2)
        def _pair(jp):
            a = 2 * jp
            _issue(a + 1, qb1, kvb1, dkb1, sq1, sk1, sd1)
            _wait_gather(qb0, kvb0, dkb0, sq0, sk0, sd0)

            @pl.when(jp > 0)
            def _d0():
                _drain_scatter(ob0, db0, dib0, sn0, sm0)
            _compute(a, qb0, kvb0, dkb0, ob0, db0, dib0)
            _scatter(a, ob0, db0, dib0, sn0, sm0)

            @pl.when(jp < IDXG // 2 - 1)
            def _pf():
                _issue(a + 2, qb0, kvb0, dkb0, sq0, sk0, sd0)
            _wait_gather(qb1, kvb1, dkb1, sq1, sk1, sd1)

            @pl.when(jp > 0)
            def _d1():
                _drain_scatter(ob1, db1, dib1, sn1, sm1)
            _compute(a + 1, qb1, kvb1, dkb1, ob1, db1, dib1)
            _scatter(a + 1, ob1, db1, dib1, sn1, sm1)

    _drain_scatter(ob0, db0, dib0, sn0, sm0)
    _drain_scatter(ob1, db1, dib1, sn1, sm1)
    plsc.subcore_barrier()
    pltpu.sync_copy(accn.at[pl.ds(s * rpn, rpn)],
                    num_hbm.at[c, pl.ds(s * rpn, rpn)])
    pltpu.sync_copy(accd.at[pl.ds(s * rpd, rpd)],
                    den_hbm.at[c, pl.ds(s * rpd, rpd)])


_edge_call = pl.kernel(
    _edge_body,
    out_type=(
        jax.ShapeDtypeStruct((NC, NP, H), _f32),
        jax.ShapeDtypeStruct((NC, NP // 8, H), _f32),
    ),
    mesh=_mesh(),
    scratch_types=[
        pltpu.VMEM_SHARED((NP, H), _f32),
        pltpu.VMEM_SHARED((NP // 8, H), _f32),
        pltpu.VMEM((IDXG, C), _i32),
        pltpu.VMEM((IDXG, C), _i32),
        pltpu.VMEM((IDXG, C), _i32),
        pltpu.VMEM((C, H), _f32),
        pltpu.VMEM((C, 2 * H), _f32),
        pltpu.VMEM((C, 2 * H), _f32),
        pltpu.VMEM((C, H), _f32),
        pltpu.VMEM((C, H), _f32),
        pltpu.VMEM((C,), _i32),
        pltpu.VMEM((C, H), _f32),
        pltpu.VMEM((C, 2 * H), _f32),
        pltpu.VMEM((C, 2 * H), _f32),
        pltpu.VMEM((C, H), _f32),
        pltpu.VMEM((C, H), _f32),
        pltpu.VMEM((C,), _i32),
        pltpu.SemaphoreType.DMA,
        pltpu.SemaphoreType.DMA,
        pltpu.SemaphoreType.DMA,
        pltpu.SemaphoreType.DMA,
        pltpu.SemaphoreType.DMA,
        pltpu.SemaphoreType.DMA,
        pltpu.SemaphoreType.DMA,
        pltpu.SemaphoreType.DMA,
        pltpu.SemaphoreType.DMA,
        pltpu.SemaphoreType.DMA,
    ],
)


def _pool_body(nr_hbm, bidx_hbm, zeros_hbm, sum_hbm, cnt_hbm,
               accs, accc, idxb, dib, nbuf, db):
    c = lax.axis_index("c")
    s = lax.axis_index("s")
    wid = s * NC + c
    rps = GP // NS
    rpc = 8

    pltpu.sync_copy(zeros_hbm.at[pl.ds(s * rps, rps)],
                    accs.at[pl.ds(s * rps, rps)])

    @pl.when(s < (GP // 8) // rpc)
    def _zc():
        pltpu.sync_copy(zeros_hbm.at[pl.ds(s * rpc, rpc)],
                        accc.at[pl.ds(s * rpc, rpc)])
    plsc.subcore_barrier()

    lane = lax.iota(_i32, 16)
    marker = jnp.where(lane == 0, 1.0, 0.0).astype(_f32)
    zero16 = jnp.zeros((16,), _f32)

    @pl.loop(0, ACH)
    def _chunk(ci):
        chunk = wid * ACH + ci
        base = wid * NODES_PT + ci * AC
        pltpu.sync_copy(bidx_hbm.at[pl.ds(chunk, 1)], idxb)
        pltpu.sync_copy(nr_hbm.at[pl.ds(base, AC)], nbuf)

        @pl.loop(0, AC)
        def _row(r):
            bvec = idxb[0, pl.ds((r >> 4) << 4, 16)]
            b7f = (bvec & 7).astype(_f32)
            bsp = _vgather(b7f, jnp.broadcast_to(r & 15, (16,)))
            for g2 in range(8):
                m = jnp.maximum(0.0, 1.0 - jnp.abs(bsp - float(g2)))
                db[r, pl.ds(g2 * 16, 16)] = marker * m

        for t in range(AC // 16):
            sl = pl.ds(t * 16, 16)
            dib[sl] = lax.shift_right_logical(idxb[0, sl], 3)

        pltpu.sync_copy(nbuf, accs.at[idxb.at[0]], add=True)
        pltpu.sync_copy(db, accc.at[dib], add=True)

    plsc.subcore_barrier()
    pltpu.sync_copy(accs.at[pl.ds(s * rps, rps)],
                    sum_hbm.at[c, pl.ds(s * rps, rps)])

    @pl.when(s < (GP // 8) // rpc)
    def _dc():
        pltpu.sync_copy(accc.at[pl.ds(s * rpc, rpc)],
                        cnt_hbm.at[c, pl.ds(s * rpc, rpc)])


_pool_call = pl.kernel(
    _pool_body,
    out_type=(
        jax.ShapeDtypeStruct((NC, GP, H), _f32),
        jax.ShapeDtypeStruct((NC, GP // 8, H), _f32),
    ),
    mesh=_mesh(),
    scratch_types=[
        pltpu.VMEM_SHARED((GP, H), _f32),
        pltpu.VMEM_SHARED((GP // 8, H), _f32),
        pltpu.VMEM((1, AC), _i32),
        pltpu.VMEM((AC,), _i32),
        pltpu.VMEM((AC, H), _f32),
        pltpu.VMEM((AC, H), _f32),
    ],
)


def _dot(a, b):
    return lax.dot_general(a, b, (((1,), (0,)), ((), ())),
                           precision=lax.Precision.HIGHEST,
                           preferred_element_type=_f32)


def _gelu(x):
    return 0.5 * x * (1.0 + lax.erf(x * (1.0 / math.sqrt(2.0))))


def _ln(x, g, b):
    mu = jnp.mean(x, axis=-1, keepdims=True)
    xc = x - mu
    var = jnp.mean(xc * xc, axis=-1, keepdims=True)
    return xc * lax.rsqrt(var + 1e-5) * g + b


def _qkv_body(x_ref, w_ref, bq_ref, nq_ref, nkv_ref):
    y = _dot(x_ref[...], w_ref[...])
    nq_ref[...] = y[:, :H] + bq_ref[...]
    nkv_ref[...] = y[:, H:]


def _qkv_call(x, w3, bq):
    return pl.pallas_call(
        _qkv_body,
        grid=(NP // BR,),
        in_specs=[
            pl.BlockSpec((BR, H), lambda i: (i, 0)),
            pl.BlockSpec((H, 3 * H), lambda i: (0, 0)),
            pl.BlockSpec((1, H), lambda i: (0, 0)),
        ],
        out_specs=[
            pl.BlockSpec((BR, H), lambda i: (i, 0)),
            pl.BlockSpec((BR, 2 * H), lambda i: (i, 0)),
        ],
        out_shape=[
            jax.ShapeDtypeStruct((NP, H), _f32),
            jax.ShapeDtypeStruct((NP, 2 * H), _f32),
        ],
    )(x, w3, bq)


def _dist_body(d_ref, wk_ref, wv_ref, bk_ref, bv_ref, out_ref):
    d = d_ref[...]
    dk = _dot(d, wk_ref[...]) + bk_ref[...]
    dv = _dot(d, wv_ref[...]) + bv_ref[...]
    out_ref[...] = jnp.concatenate([dk, dv], axis=1)


def _dist_call(demb, wk, wv, bk, bv):
    return pl.pallas_call(
        _dist_body,
        out_shape=jax.ShapeDtypeStruct((DP, 2 * H), _f32),
    )(demb, wk, wv, bk, bv)


def _upd_body(num_ref, den_ref, nr_ref, wa_ref, ba_ref, g1_ref, b1_ref,
              wm_ref, bm_ref, wo_ref, bo_ref, g2_ref, b2_ref, out_ref):
    num = num_ref[0] + num_ref[1]
    den8 = den_ref[0][:, :8] + den_ref[1][:, :8]
    r8 = lax.broadcasted_iota(_i32, (8, H), 0)
    c128 = lax.broadcasted_iota(_i32, (8, H), 1) // DKH
    sel = (r8 == c128).astype(_f32)
    den = _dot(den8, sel)
    aggr = num / (den + 1e-16)
    a = _dot(_gelu(aggr), wa_ref[...]) + ba_ref[...] + nr_ref[...]
    trans = _ln(a, g1_ref[...], b1_ref[...])
    mid = _gelu(_dot(trans, wm_ref[...]) + bm_ref[...])
    nr2 = _ln(_dot(mid, wo_ref[...]) + bo_ref[...] + trans,
              g2_ref[...], b2_ref[...])
    rows = pl.program_id(0) * BR + lax.broadcasted_iota(_i32, (BR, H), 0)
    out_ref[...] = jnp.where(rows < N, nr2, 0.0)


def _upd_call(num, den, nr, wa, ba, g1, b1, wm, bm, wo, bo, g2, b2):
    vec = lambda i: (0, 0)
    return pl.pallas_call(
        _upd_body,
        grid=(NP // BR,),
        in_specs=[
            pl.BlockSpec((2, BR, H), lambda i: (0, i, 0)),
            pl.BlockSpec((2, BR, 16), lambda i: (0, i, 0)),
            pl.BlockSpec((BR, H), lambda i: (i, 0)),
            pl.BlockSpec((H, H), vec),
            pl.BlockSpec((1, H), vec),
            pl.BlockSpec((1, H), vec),
            pl.BlockSpec((1, H), vec),
            pl.BlockSpec((H, 2 * H), vec),
            pl.BlockSpec((1, 2 * H), vec),
            pl.BlockSpec((2 * H, H), vec),
            pl.BlockSpec((1, H), vec),
            pl.BlockSpec((1, H), vec),
            pl.BlockSpec((1, H), vec),
        ],
        out_specs=pl.BlockSpec((BR, H), lambda i: (i, 0)),
        out_shape=jax.ShapeDtypeStruct((NP, H), _f32),
    )(num, den, nr, wa, ba, g1, b1, wm, bm, wo, bo, g2, b2)


def _final_body(ps_ref, pc_ref, wout_ref, bout_ref, out_ref):
    sums = ps_ref[0][:G, :] + ps_ref[1][:G, :]
    cnt = pc_ref[0][:G, :1] + pc_ref[1][:G, :1]
    mean = sums / jnp.maximum(cnt, 1.0)
    out_ref[...] = _dot(mean, wout_ref[...]) + bout_ref[...]


def _final_call(psum, pcnt, wout, bout):
    return pl.pallas_call(
        _final_body,
        out_shape=jax.ShapeDtypeStruct((G, H), _f32),
    )(psum, pcnt, wout, bout)


def kernel(node_attr, batch_idx, edge_index, strats, params):
    p = params
    attr = node_attr.astype(_i32)
    attr_p = jnp.pad(attr, ((0, NP - N), (0, 0)))
    aidx = (attr_p + (jnp.arange(NF, dtype=_i32) * VOCAB)[None, :]).T
    aidx = aidx.reshape(NF, NW * ACH, AC).transpose(1, 0, 2)
    emb2d = p['atom_emb'].reshape(NF * VOCAB, H)

    node_rep = _atom_call(emb2d, aidx)

    src = edge_index[0].astype(_i32)
    dst = edge_index[1].astype(_i32)
    st = strats.astype(_i32)
    padn = EP - E
    srcp = jnp.concatenate([src, jnp.zeros((padn,), _i32)]).reshape(EP // C, C)
    dstp = jnp.concatenate([dst, jnp.full((padn,), NP - 1, _i32)]).reshape(EP // C, C)
    stp = jnp.concatenate([st, jnp.zeros((padn,), _i32)]).reshape(EP // C, C)
    zeros = jnp.zeros((NP, H), _f32)

    for l in range(2):
        w3 = jnp.concatenate([p['Wq'][l], p['Wk'][l], p['Wv'][l]], axis=1)
        nq, nkv = _qkv_call(node_rep, w3, p['bq'][l].reshape(1, H))
        demb = jnp.pad(p['dist_emb'][l], ((0, DP - MAXD), (0, 0)))
        dkv = _dist_call(demb, p['Wk'][l], p['Wv'][l],
                         p['bk'][l].reshape(1, H), p['bv'][l].reshape(1, H))
        num, den = _edge_call(nq, nkv, dkv, srcp, dstp, stp, zeros)
        den = den.reshape(NC, NP, 16)
        node_rep = _upd_call(
            num, den, node_rep,
            p['Wa'][l], p['ba'][l].reshape(1, H),
            p['ln1_g'][l].reshape(1, H), p['ln1_b'][l].reshape(1, H),
            p['Wmid'][l], p['bmid'][l].reshape(1, 2 * H),
            p['Wo2'][l], p['bo2'][l].reshape(1, H),
            p['ln2_g'][l].reshape(1, H), p['ln2_b'][l].reshape(1, H))

    bidx = jnp.concatenate([batch_idx.astype(_i32), jnp.full((NP - N,), G, _i32)])
    bidx = bidx.reshape(NW * ACH, AC)
    psum, pcnt = _pool_call(node_rep, bidx, zeros)
    pcnt = pcnt.reshape(NC, GP, 16)
    return _final_call(psum, pcnt, p['Wout'], p['bout'].reshape(1, H))
